# baseline (speedup 1.0000x reference)
import functools

import jax
import jax.numpy as jnp
from jax import lax
from jax.experimental import pallas as pl
from jax.experimental.pallas import tpu as pltpu
from jax.experimental.pallas import tpu_sc as plsc

N_NODES = 10000
D = 128
E_EDGES = 320000

NCORES = 2
NSUB = 16
NTILES = NCORES * NSUB

NP = 10240
ROWS_PER_TILE = NP // NSUB

CHUNK = 128
K0 = 80
K1 = 80
KMAX = max(K0, K1)
EPAD = NSUB * (K0 + K1) * CHUNK

BLK = 2048

_mesh = plsc.VectorSubcoreMesh(core_axis_name="c", subcore_axis_name="s")


@functools.partial(
    pl.kernel,
    out_type=jax.ShapeDtypeStruct((NCORES, NP), jnp.float32),
    mesh=_mesh,
    scratch_types=[
        pltpu.VMEM((KMAX, CHUNK), jnp.int32),
        pltpu.VMEM((CHUNK,), jnp.float32),
        pltpu.VMEM((ROWS_PER_TILE,), jnp.float32),
        pltpu.VMEM_SHARED((NP,), jnp.float32),
    ],
)
def _degree_kernel(dst_hbm, ones_hbm, zeros_hbm, out_hbm, idst, ones_v,
                   zeros_v, acc):
    cid = lax.axis_index("c")
    sid = lax.axis_index("s")
    base = sid * ROWS_PER_TILE
    pltpu.sync_copy(dst_hbm.at[cid, sid], idst)
    pltpu.sync_copy(ones_hbm, ones_v)
    pltpu.sync_copy(zeros_hbm.at[pl.ds(base, ROWS_PER_TILE)], zeros_v)

    pltpu.sync_copy(zeros_v, acc.at[pl.ds(base, ROWS_PER_TILE)])
    plsc.subcore_barrier()

    def body(j, carry):
        pltpu.sync_copy(ones_v, acc.at[idst.at[j]], add=True)
        return carry

    @pl.when(cid == 0)
    def _():
        lax.fori_loop(0, K0, body, 0)

    @pl.when(cid != 0)
    def _():
        lax.fori_loop(0, K1, body, 0)
    plsc.subcore_barrier()
    pltpu.sync_copy(acc.at[pl.ds(base, ROWS_PER_TILE)],
                    out_hbm.at[cid, pl.ds(base, ROWS_PER_TILE)])


@functools.partial(
    pl.kernel,
    out_type=jax.ShapeDtypeStruct((NCORES, NP, D), jnp.float32),
    mesh=_mesh,
    scratch_types=[
        pltpu.VMEM((KMAX, CHUNK), jnp.int32),
        pltpu.VMEM((KMAX, CHUNK), jnp.int32),
        pltpu.VMEM((CHUNK, D), jnp.float32),
        pltpu.VMEM_SHARED((NP, D), jnp.float32),
        pltpu.SemaphoreType.DMA,
    ],
)
def _scatter_kernel(y_hbm, src_hbm, dst_hbm, z_hbm, out_hbm, isrc, idst,
                    rows, acc, sem):
    cid = lax.axis_index("c")
    sid = lax.axis_index("s")
    base = sid * ROWS_PER_TILE
    pltpu.sync_copy(src_hbm.at[cid, sid], isrc)
    pltpu.sync_copy(dst_hbm.at[cid, sid], idst)

    pltpu.sync_copy(z_hbm.at[pl.ds(base, ROWS_PER_TILE)],
                    acc.at[pl.ds(base, ROWS_PER_TILE)])
    plsc.subcore_barrier()

    def body(j, carry):
        pltpu.async_copy(y_hbm.at[isrc.at[j]], rows, sem).wait()
        pltpu.sync_copy(rows, acc.at[idst.at[j]], add=True)
        return carry

    @pl.when(cid == 0)
    def _():
        lax.fori_loop(0, K0, body, 0)

    @pl.when(cid != 0)
    def _():
        lax.fori_loop(0, K1, body, 0)
    plsc.subcore_barrier()
    pltpu.sync_copy(acc.at[pl.ds(base, ROWS_PER_TILE)],
                    out_hbm.at[cid, pl.ds(base, ROWS_PER_TILE)])


def _tc_first(d0, d1, x_p, W):
    def kfn(d0_ref, d1_ref, x_ref, w_ref, y_ref):
        s = lax.rsqrt(d0_ref[...] + d1_ref[...] + 1.0)
        h = jnp.dot(x_ref[...], w_ref[...], preferred_element_type=jnp.float32)
        y_ref[...] = h * s

    return pl.pallas_call(
        kfn,
        grid=(NP // BLK,),
        in_specs=[
            pl.BlockSpec((BLK, 1), lambda i: (i, 0)),
            pl.BlockSpec((BLK, 1), lambda i: (i, 0)),
            pl.BlockSpec((BLK, D), lambda i: (i, 0)),
            pl.BlockSpec((D, D), lambda i: (0, 0)),
        ],
        out_specs=pl.BlockSpec((BLK, D), lambda i: (i, 0)),
        out_shape=jax.ShapeDtypeStruct((NP, D), jnp.float32),
    )(d0, d1, x_p, W)


def _tc_mid(d0, d1, a0, a1, y, b, W):
    def kfn(d0_ref, d1_ref, a0_ref, a1_ref, y_ref, b_ref, w_ref, o_ref):
        s = lax.rsqrt(d0_ref[...] + d1_ref[...] + 1.0)
        z = (a0_ref[...] + a1_ref[...] + y_ref[...]) * s + b_ref[...]
        z = jnp.maximum(z, 0.0)
        h = jnp.dot(z, w_ref[...], preferred_element_type=jnp.float32)
        o_ref[...] = h * s

    return pl.pallas_call(
        kfn,
        grid=(NP // BLK,),
        in_specs=[
            pl.BlockSpec((BLK, 1), lambda i: (i, 0)),
            pl.BlockSpec((BLK, 1), lambda i: (i, 0)),
            pl.BlockSpec((BLK, D), lambda i: (i, 0)),
            pl.BlockSpec((BLK, D), lambda i: (i, 0)),
            pl.BlockSpec((BLK, D), lambda i: (i, 0)),
            pl.BlockSpec((1, D), lambda i: (0, 0)),
            pl.BlockSpec((D, D), lambda i: (0, 0)),
        ],
        out_specs=pl.BlockSpec((BLK, D), lambda i: (i, 0)),
        out_shape=jax.ShapeDtypeStruct((NP, D), jnp.float32),
    )(d0, d1, a0, a1, y, b, W)


def _tc_last(d0, d1, a0, a1, y, b):
    def kfn(d0_ref, d1_ref, a0_ref, a1_ref, y_ref, b_ref, o_ref):
        s = lax.rsqrt(d0_ref[...] + d1_ref[...] + 1.0)
        o_ref[...] = (a0_ref[...] + a1_ref[...] + y_ref[...]) * s + b_ref[...]

    return pl.pallas_call(
        kfn,
        grid=(NP // BLK,),
        in_specs=[
            pl.BlockSpec((BLK, 1), lambda i: (i, 0)),
            pl.BlockSpec((BLK, 1), lambda i: (i, 0)),
            pl.BlockSpec((BLK, D), lambda i: (i, 0)),
            pl.BlockSpec((BLK, D), lambda i: (i, 0)),
            pl.BlockSpec((BLK, D), lambda i: (i, 0)),
            pl.BlockSpec((1, D), lambda i: (0, 0)),
        ],
        out_specs=pl.BlockSpec((BLK, D), lambda i: (i, 0)),
        out_shape=jax.ShapeDtypeStruct((NP, D), jnp.float32),
    )(d0, d1, a0, a1, y, b)


def kernel(x, edge_index, W1, b1, W2, b2):
    n = x.shape[0]
    pad_e = EPAD - E_EDGES
    src = jnp.concatenate([edge_index[0], jnp.full((pad_e,), n, jnp.int32)])
    dst = jnp.concatenate([edge_index[1], jnp.full((pad_e,), n, jnp.int32)])

    def split(a):
        e0 = NSUB * K0 * CHUNK
        a0 = a[:e0].reshape(NSUB, K0, CHUNK)
        a1 = a[e0:].reshape(NSUB, K1, CHUNK)
        full = jnp.full((NCORES, NSUB, KMAX, CHUNK), n, jnp.int32)
        return full.at[0, :, :K0].set(a0).at[1, :, :K1].set(a1)

    src5 = split(src)
    dst5 = split(dst)

    x_p = jnp.pad(x, ((0, NP - n), (0, 0)))
    ones_c = jnp.ones((CHUNK,), jnp.float32)
    zeros_np = jnp.zeros((NP,), jnp.float32)
    zeros_nd = jnp.zeros((NP, D), jnp.float32)

    degp = _degree_kernel(dst5, ones_c, zeros_np)
    d0 = degp[0].reshape(NP, 1)
    d1 = degp[1].reshape(NP, 1)

    b1r = b1.reshape(1, D)
    b2r = b2.reshape(1, D)

    y1 = _tc_first(d0, d1, x_p, W1)
    acc1 = _scatter_kernel(y1, src5, dst5, zeros_nd)
    y2 = _tc_mid(d0, d1, acc1[0], acc1[1], y1, b1r, W2)
    acc2 = _scatter_kernel(y2, src5, dst5, zeros_nd)
    out = _tc_last(d0, d1, acc2[0], acc2[1], y2, b2r)
    return out[:n]


# speedup vs baseline: 1.2670x; 1.2670x over previous
import functools

import jax
import jax.numpy as jnp
from jax import lax
from jax.experimental import pallas as pl
from jax.experimental.pallas import tpu as pltpu
from jax.experimental.pallas import tpu_sc as plsc

N_NODES = 10000
D = 128
E_EDGES = 320000

NCORES = 2
NSUB = 16
NTILES = NCORES * NSUB

NP = 10240
ROWS_PER_TILE = NP // NSUB

CHUNK = 128
NCHUNK = 79
EPAD = NTILES * NCHUNK * CHUNK

BLK = 2048

_mesh = plsc.VectorSubcoreMesh(core_axis_name="c", subcore_axis_name="s")


@functools.partial(
    pl.kernel,
    out_type=jax.ShapeDtypeStruct((NCORES, NP), jnp.float32),
    mesh=_mesh,
    scratch_types=[
        pltpu.VMEM((NCHUNK, CHUNK), jnp.int32),
        pltpu.VMEM((CHUNK,), jnp.float32),
        pltpu.VMEM((ROWS_PER_TILE,), jnp.float32),
        pltpu.VMEM_SHARED((NP,), jnp.float32),
    ],
)
def _degree_kernel(dst_hbm, ones_hbm, zeros_hbm, out_hbm, idst, ones_v,
                   zeros_v, acc):
    cid = lax.axis_index("c")
    sid = lax.axis_index("s")
    wid = sid * NCORES + cid
    base = sid * ROWS_PER_TILE
    pltpu.sync_copy(dst_hbm.at[wid], idst)
    pltpu.sync_copy(ones_hbm, ones_v)
    pltpu.sync_copy(zeros_hbm.at[pl.ds(base, ROWS_PER_TILE)], zeros_v)

    pltpu.sync_copy(zeros_v, acc.at[pl.ds(base, ROWS_PER_TILE)])
    plsc.subcore_barrier()

    def body(j, carry):
        pltpu.sync_copy(ones_v, acc.at[idst.at[j]], add=True)
        return carry

    lax.fori_loop(0, NCHUNK, body, 0)
    plsc.subcore_barrier()
    pltpu.sync_copy(acc.at[pl.ds(base, ROWS_PER_TILE)],
                    out_hbm.at[cid, pl.ds(base, ROWS_PER_TILE)])


@functools.partial(
    pl.kernel,
    out_type=jax.ShapeDtypeStruct((NCORES, NP, D), jnp.float32),
    mesh=_mesh,
    scratch_types=[
        pltpu.VMEM((NCHUNK, CHUNK), jnp.int32),
        pltpu.VMEM((NCHUNK, CHUNK), jnp.int32),
        pltpu.VMEM((CHUNK, D), jnp.float32),
        pltpu.VMEM_SHARED((NP, D), jnp.float32),
        pltpu.SemaphoreType.DMA,
    ],
)
def _scatter_kernel(y_hbm, src_hbm, dst_hbm, z_hbm, out_hbm, isrc, idst, rows,
                    acc, sem):
    cid = lax.axis_index("c")
    sid = lax.axis_index("s")
    wid = sid * NCORES + cid
    base = sid * ROWS_PER_TILE
    pltpu.sync_copy(src_hbm.at[wid], isrc)
    pltpu.sync_copy(dst_hbm.at[wid], idst)

    @pl.when(cid == 0)
    def _():
        pltpu.sync_copy(y_hbm.at[pl.ds(base, ROWS_PER_TILE)],
                        acc.at[pl.ds(base, ROWS_PER_TILE)])

    @pl.when(cid != 0)
    def _():
        pltpu.sync_copy(z_hbm.at[pl.ds(base, ROWS_PER_TILE)],
                        acc.at[pl.ds(base, ROWS_PER_TILE)])
    plsc.subcore_barrier()

    def body(j, carry):
        pltpu.async_copy(y_hbm.at[isrc.at[j]], rows, sem).wait()
        pltpu.sync_copy(rows, acc.at[idst.at[j]], add=True)
        return carry

    lax.fori_loop(0, NCHUNK, body, 0)
    plsc.subcore_barrier()
    pltpu.sync_copy(acc.at[pl.ds(base, ROWS_PER_TILE)],
                    out_hbm.at[cid, pl.ds(base, ROWS_PER_TILE)])


def _tc_first(d0, d1, x_p, W):
    def kfn(d0_ref, d1_ref, x_ref, w_ref, y_ref):
        s = lax.rsqrt(d0_ref[...] + d1_ref[...] + 1.0)
        h = jnp.dot(x_ref[...], w_ref[...], preferred_element_type=jnp.float32)
        y_ref[...] = h * s

    return pl.pallas_call(
        kfn,
        grid=(NP // BLK,),
        in_specs=[
            pl.BlockSpec((BLK, 1), lambda i: (i, 0)),
            pl.BlockSpec((BLK, 1), lambda i: (i, 0)),
            pl.BlockSpec((BLK, D), lambda i: (i, 0)),
            pl.BlockSpec((D, D), lambda i: (0, 0)),
        ],
        out_specs=pl.BlockSpec((BLK, D), lambda i: (i, 0)),
        out_shape=jax.ShapeDtypeStruct((NP, D), jnp.float32),
    )(d0, d1, x_p, W)


def _tc_mid(d0, d1, a0, a1, b, W):
    def kfn(d0_ref, d1_ref, a0_ref, a1_ref, b_ref, w_ref, y_ref):
        s = lax.rsqrt(d0_ref[...] + d1_ref[...] + 1.0)
        z = (a0_ref[...] + a1_ref[...]) * s + b_ref[...]
        z = jnp.maximum(z, 0.0)
        h = jnp.dot(z, w_ref[...], preferred_element_type=jnp.float32)
        y_ref[...] = h * s

    return pl.pallas_call(
        kfn,
        grid=(NP // BLK,),
        in_specs=[
            pl.BlockSpec((BLK, 1), lambda i: (i, 0)),
            pl.BlockSpec((BLK, 1), lambda i: (i, 0)),
            pl.BlockSpec((BLK, D), lambda i: (i, 0)),
            pl.BlockSpec((BLK, D), lambda i: (i, 0)),
            pl.BlockSpec((1, D), lambda i: (0, 0)),
            pl.BlockSpec((D, D), lambda i: (0, 0)),
        ],
        out_specs=pl.BlockSpec((BLK, D), lambda i: (i, 0)),
        out_shape=jax.ShapeDtypeStruct((NP, D), jnp.float32),
    )(d0, d1, a0, a1, b, W)


def _tc_last(d0, d1, a0, a1, b):
    def kfn(d0_ref, d1_ref, a0_ref, a1_ref, b_ref, o_ref):
        s = lax.rsqrt(d0_ref[...] + d1_ref[...] + 1.0)
        o_ref[...] = (a0_ref[...] + a1_ref[...]) * s + b_ref[...]

    return pl.pallas_call(
        kfn,
        grid=(NP // BLK,),
        in_specs=[
            pl.BlockSpec((BLK, 1), lambda i: (i, 0)),
            pl.BlockSpec((BLK, 1), lambda i: (i, 0)),
            pl.BlockSpec((BLK, D), lambda i: (i, 0)),
            pl.BlockSpec((BLK, D), lambda i: (i, 0)),
            pl.BlockSpec((1, D), lambda i: (0, 0)),
        ],
        out_specs=pl.BlockSpec((BLK, D), lambda i: (i, 0)),
        out_shape=jax.ShapeDtypeStruct((NP, D), jnp.float32),
    )(d0, d1, a0, a1, b)


def kernel(x, edge_index, W1, b1, W2, b2):
    n = x.shape[0]
    pad_e = EPAD - E_EDGES
    src = jnp.concatenate([edge_index[0], jnp.full((pad_e,), n, jnp.int32)])
    dst = jnp.concatenate([edge_index[1], jnp.full((pad_e,), n, jnp.int32)])
    src3 = src.reshape(NTILES, NCHUNK, CHUNK)
    dst3 = dst.reshape(NTILES, NCHUNK, CHUNK)

    x_p = jnp.pad(x, ((0, NP - n), (0, 0)))
    ones_c = jnp.ones((CHUNK,), jnp.float32)
    zeros_np = jnp.zeros((NP,), jnp.float32)
    zeros_nd = jnp.zeros((NP, D), jnp.float32)

    degp = _degree_kernel(dst3, ones_c, zeros_np)
    d0 = degp[0].reshape(NP, 1)
    d1 = degp[1].reshape(NP, 1)

    b1r = b1.reshape(1, D)
    b2r = b2.reshape(1, D)

    y1 = _tc_first(d0, d1, x_p, W1)
    acc1 = _scatter_kernel(y1, src3, dst3, zeros_nd)
    y2 = _tc_mid(d0, d1, acc1[0], acc1[1], b1r, W2)
    acc2 = _scatter_kernel(y2, src3, dst3, zeros_nd)
    out = _tc_last(d0, d1, acc2[0], acc2[1], b2r)
    return out[:n]
